# Initial kernel scaffold; baseline (speedup 1.0000x reference)
import jax
import jax.numpy as jnp
from jax.experimental import pallas as pl


def kernel(x, centroids, centroid_labels):
    raise NotImplementedError("write your pallas kernel here")


# speedup vs baseline: 11.9112x; 11.9112x over previous
import jax
import jax.numpy as jnp
from jax.experimental import pallas as pl
from jax.experimental.pallas import tpu as pltpu

_BN = 1024
_BK = 1024
_BNORM = 2048


def _norm_body(x_ref, o_ref):
    x = x_ref[...]
    n = jnp.sqrt(jnp.sum(x * x, axis=1, keepdims=True))
    o_ref[...] = (x / (n + 1e-12)).astype(jnp.bfloat16)


def _normalize_bf16(a):
    rows, d = a.shape
    bn = min(_BNORM, rows)
    return pl.pallas_call(
        _norm_body,
        grid=(rows // bn,),
        in_specs=[pl.BlockSpec((bn, d), lambda i: (i, 0))],
        out_specs=pl.BlockSpec((bn, d), lambda i: (i, 0)),
        out_shape=jax.ShapeDtypeStruct((rows, d), jnp.bfloat16),
    )(a)


def _main_body(lab_ref, xn_ref, cn_ref, out_ref, *, nkt, n_classes):
    j = pl.program_id(1)

    @pl.when(j == 0)
    def _():
        out_ref[...] = jnp.zeros_like(out_ref)

    sim = jax.lax.dot_general(
        xn_ref[...], cn_ref[...], (((1,), (1,)), ((), ())),
        preferred_element_type=jnp.float32)
    e = jnp.exp(sim).astype(jnp.bfloat16)
    lab = lab_ref[0]
    bk = lab.shape[1]
    oh_t = (jax.lax.broadcasted_iota(jnp.int32, (n_classes, bk), 0)
            == lab).astype(jnp.bfloat16)
    out_ref[...] += jax.lax.dot_general(
        e, oh_t, (((1,), (1,)), ((), ())),
        preferred_element_type=jnp.float32)

    @pl.when(j == nkt - 1)
    def _():
        acc = out_ref[...]
        out_ref[...] = acc / jnp.sum(acc, axis=1, keepdims=True)


def kernel(x, centroids, centroid_labels):
    n, d = x.shape
    k = centroids.shape[0]
    labels = centroid_labels.astype(jnp.int32)
    n_classes = 1024

    xn = _normalize_bf16(x)
    cn = _normalize_bf16(centroids)

    bn, bk = min(_BN, n), min(_BK, k)
    nkt = k // bk
    lab3 = labels.reshape(nkt, 1, bk)

    import functools
    body = functools.partial(_main_body, nkt=nkt, n_classes=n_classes)
    return pl.pallas_call(
        body,
        grid=(n // bn, nkt),
        in_specs=[
            pl.BlockSpec((1, 1, bk), lambda i, j: (j, 0, 0)),
            pl.BlockSpec((bn, d), lambda i, j: (i, 0)),
            pl.BlockSpec((bk, d), lambda i, j: (j, 0)),
        ],
        out_specs=pl.BlockSpec((bn, n_classes), lambda i, j: (i, 0)),
        out_shape=jax.ShapeDtypeStruct((n, n_classes), jnp.float32),
        compiler_params=pltpu.CompilerParams(
            dimension_semantics=("parallel", "arbitrary")),
    )(lab3, xn, cn)
